# baseline (speedup 1.0000x reference)
import jax
import jax.numpy as jnp
from jax import lax
from jax.experimental import pallas as pl
from jax.experimental.pallas import tpu as pltpu
from jax.experimental.pallas import tpu_sc as plsc

_SC_MESH = plsc.VectorSubcoreMesh(core_axis_name="c", subcore_axis_name="s")
_NC = 2
_NS = 16
_NW = _NC * _NS

E = 64
K = 64
D = 1024
B_T = 4096

_ROWS = 512
_GRID1 = B_T // _ROWS


def _gate_body(x2d_ref, gw_ref, sv_ref, si_ref, cnt_ref, sc_ref):
    i = pl.program_id(0)
    s = jnp.dot(x2d_ref[...], gw_ref[...], preferred_element_type=jnp.float32)
    sc_ref[pl.ds(i * _ROWS, _ROWS), :] = s

    @pl.when(i == _GRID1 - 1)
    def _():
        riota = lax.broadcasted_iota(jnp.int32, (B_T, E), 0)

        def body(a, carry):
            sc, cnt = carry
            m = jnp.max(sc, axis=0)
            ismax = sc == m[None, :]
            idx = jnp.min(jnp.where(ismax, riota, B_T), axis=0)
            chosen = riota == idx[None, :]
            cnt = cnt + chosen.astype(jnp.float32)
            sc = jnp.where(chosen, -jnp.inf, sc)
            sv_ref[pl.ds(a, 1), :] = m.reshape(1, E)
            si_ref[pl.ds(a, 1), :] = idx.reshape(1, E)
            return sc, cnt

        init = (sc_ref[...], jnp.zeros((B_T, E), jnp.float32))
        _, cnt = lax.fori_loop(0, K, body, init)
        cnt_ref[...] = jnp.sum(cnt, axis=1, keepdims=True)


def _gate_topk(x2d, gw):
    return pl.pallas_call(
        _gate_body,
        grid=(_GRID1,),
        in_specs=[
            pl.BlockSpec((_ROWS, 2 * D), lambda i: (i, 0)),
            pl.BlockSpec((2 * D, E), lambda i: (0, 0)),
        ],
        out_specs=[
            pl.BlockSpec((K, E), lambda i: (0, 0)),
            pl.BlockSpec((K, E), lambda i: (0, 0)),
            pl.BlockSpec((B_T, 1), lambda i: (0, 0)),
        ],
        out_shape=[
            jax.ShapeDtypeStruct((K, E), jnp.float32),
            jax.ShapeDtypeStruct((K, E), jnp.int32),
            jax.ShapeDtypeStruct((B_T, 1), jnp.float32),
        ],
        scratch_shapes=[pltpu.VMEM((B_T, E), jnp.float32)],
    )(x2d, gw)


_GROWS = B_T // _NW
_GCHUNK = 64


def _sc_gather(xpk, flat):
    def body(x_hbm, i_hbm, o_hbm, idx_v, buf, sem):
        wid = lax.axis_index("c") * _NS + lax.axis_index("s")
        base = wid * _GROWS
        pltpu.sync_copy(i_hbm.at[pl.ds(base, _GROWS)], idx_v)
        for c2 in range(_GROWS // _GCHUNK):
            pltpu.async_copy(
                x_hbm.at[idx_v.at[pl.ds(c2 * _GCHUNK, _GCHUNK)]], buf, sem
            ).wait()
            pltpu.sync_copy(buf, o_hbm.at[pl.ds(base + c2 * _GCHUNK, _GCHUNK)])

    k = pl.kernel(
        body,
        out_type=jax.ShapeDtypeStruct((B_T, D), jnp.int32),
        mesh=_SC_MESH,
        scratch_types=[
            pltpu.VMEM((_GROWS,), jnp.int32),
            pltpu.VMEM((_GCHUNK, D), jnp.int32),
            pltpu.SemaphoreType.DMA,
        ],
    )
    return k(xpk, flat)


_KC = 512
_HC = 512

_INV_SQRT2 = 0.7071067811865476


def _gelu_exact(v):
    return 0.5 * v * (1.0 + lax.erf(v * _INV_SQRT2))


def _combine_body(y_ref, flat_ref, cnt_ref, bias_ref, res_ref, out_scr):
    k = pl.program_id(1)
    tok_row = flat_ref[0]
    oh = (lax.broadcasted_iota(jnp.int32, (B_T, _KC), 0) == tok_row).astype(
        jnp.bfloat16
    )
    yb = y_ref[...].astype(jnp.bfloat16)
    part = jnp.dot(oh, yb, preferred_element_type=jnp.float32)

    @pl.when(k == 0)
    def _():
        out_scr[...] = part

    @pl.when(k > 0)
    def _():
        out_scr[...] += part

    @pl.when(k == B_T // _KC - 1)
    def _():
        cnt = jnp.clip(cnt_ref[...], 1.0, None)
        res_ref[...] = _gelu_exact(out_scr[...] / cnt + bias_ref[...])


def _combine(y_all, flat3, counts, bias_int):
    return pl.pallas_call(
        _combine_body,
        grid=(2 * D // _HC, B_T // _KC),
        in_specs=[
            pl.BlockSpec((_KC, _HC), lambda h, k: (k, h)),
            pl.BlockSpec((1, 1, _KC), lambda h, k: (k, 0, 0)),
            pl.BlockSpec((B_T, 1), lambda h, k: (0, 0)),
            pl.BlockSpec((1, _HC), lambda h, k: (0, h)),
        ],
        out_specs=pl.BlockSpec((B_T, _HC), lambda h, k: (0, h)),
        out_shape=jax.ShapeDtypeStruct((B_T, 2 * D), jnp.float32),
        scratch_shapes=[pltpu.VMEM((B_T, _HC), jnp.float32)],
    )(y_all, flat3, counts, bias_int)


def _expert_body(xg_ref, w_ref, s_ref, y_ref):
    blk = xg_ref[...]
    xc = jnp.concatenate([blk[:, :D], blk[:, D:]], axis=0)
    w = w_ref[...]
    ab = jnp.dot(xc, w, preferred_element_type=jnp.float32)
    a = ab[:K]
    b = ab[K:]
    rm1 = pltpu.roll(b, 2 * D - 1, axis=1)
    r1 = pltpu.roll(b, 1, axis=1)
    lane = lax.broadcasted_iota(jnp.int32, (K, 2 * D), 1)
    bswap = jnp.where(lane % 2 == 0, -rm1, r1)
    y_ref[...] = (a + bswap) * s_ref[...]


def _experts(xg, w3, sflat):
    return pl.pallas_call(
        _expert_body,
        grid=(E,),
        in_specs=[
            pl.BlockSpec((K, 2 * D), lambda a: (a, 0)),
            pl.BlockSpec((D, 2 * D), lambda a: (a, 0)),
            pl.BlockSpec((K, 1), lambda a: (a, 0)),
        ],
        out_specs=pl.BlockSpec((K, 2 * D), lambda a: (a, 0)),
        out_shape=jax.ShapeDtypeStruct((B_T, 2 * D), jnp.float32),
    )(xg, w3, sflat)


def kernel(x, gate_weights, experts_weight, act_bias):
    x2d = x.reshape(B_T, 2 * D)
    xcat = jnp.concatenate(
        [x[:, :, 0].astype(jnp.bfloat16), x[:, :, 1].astype(jnp.bfloat16)],
        axis=1,
    )
    xpk = lax.bitcast_convert_type(xcat.reshape(B_T, D, 2), jnp.int32)
    w3 = experts_weight.reshape(E * D, 2 * D).astype(jnp.bfloat16)

    sv, si, counts = _gate_topk(x2d, gate_weights)
    topk_scores = sv.T
    topk_indices = si.T
    flat = si.reshape(-1)

    xg = lax.bitcast_convert_type(
        _sc_gather(xpk, flat), jnp.bfloat16
    ).reshape(B_T, 2 * D)

    y_all = _experts(xg, w3, sv.reshape(B_T, 1))

    res2d = _combine(
        y_all,
        flat.reshape(B_T // _KC, 1, _KC),
        counts,
        jnp.repeat(act_bias, 2).reshape(1, 2 * D),
    )
    res = res2d.reshape(B_T, D, 2)
    return (res, topk_indices, topk_scores, counts.reshape(B_T, 1, 1))


# speedup vs baseline: 1.1392x; 1.1392x over previous
import jax
import jax.numpy as jnp
from jax import lax
from jax.experimental import pallas as pl
from jax.experimental.pallas import tpu as pltpu
from jax.experimental.pallas import tpu_sc as plsc

_SC_MESH = plsc.VectorSubcoreMesh(core_axis_name="c", subcore_axis_name="s")
_NC = 2
_NS = 16
_NW = _NC * _NS

E = 64
K = 64
D = 1024
B_T = 4096

_ROWS = 512
_GRID1 = B_T // _ROWS


def _gate_body(x2d_ref, gw_ref, sv_ref, si_ref, cnt_ref, sc_ref):
    i = pl.program_id(0)
    s = jnp.dot(x2d_ref[...], gw_ref[...], preferred_element_type=jnp.float32)
    sc_ref[pl.ds(i * _ROWS, _ROWS), :] = s

    @pl.when(i == _GRID1 - 1)
    def _():
        riota = lax.broadcasted_iota(jnp.int32, (B_T, E), 0)

        def body(a, carry):
            sc, cnt = carry
            m = jnp.max(sc, axis=0)
            ismax = sc == m[None, :]
            idx = jnp.min(jnp.where(ismax, riota, B_T), axis=0)
            chosen = riota == idx[None, :]
            cnt = cnt + chosen.astype(jnp.float32)
            sc = jnp.where(chosen, -jnp.inf, sc)
            sv_ref[pl.ds(a, 1), :] = m.reshape(1, E)
            si_ref[pl.ds(a, 1), :] = idx.reshape(1, E)
            return sc, cnt

        init = (sc_ref[...], jnp.zeros((B_T, E), jnp.float32))
        _, cnt = lax.fori_loop(0, K, body, init)
        cnt_ref[...] = jnp.sum(cnt, axis=1, keepdims=True)


def _gate_topk(x2d, gw):
    return pl.pallas_call(
        _gate_body,
        grid=(_GRID1,),
        in_specs=[
            pl.BlockSpec((_ROWS, 2 * D), lambda i: (i, 0)),
            pl.BlockSpec((2 * D, E), lambda i: (0, 0)),
        ],
        out_specs=[
            pl.BlockSpec((K, E), lambda i: (0, 0)),
            pl.BlockSpec((K, E), lambda i: (0, 0)),
            pl.BlockSpec((B_T, 1), lambda i: (0, 0)),
        ],
        out_shape=[
            jax.ShapeDtypeStruct((K, E), jnp.float32),
            jax.ShapeDtypeStruct((K, E), jnp.int32),
            jax.ShapeDtypeStruct((B_T, 1), jnp.float32),
        ],
        scratch_shapes=[pltpu.VMEM((B_T, E), jnp.float32)],
    )(x2d, gw)


_GROWS = B_T // _NW
_GCHUNK = 64


def _sc_gather(xpk, flat):
    def body(x_hbm, i_hbm, o_hbm, idx_v, buf, sem):
        wid = lax.axis_index("c") * _NS + lax.axis_index("s")
        base = wid * _GROWS
        pltpu.sync_copy(i_hbm.at[pl.ds(base, _GROWS)], idx_v)
        for c2 in range(_GROWS // _GCHUNK):
            pltpu.async_copy(
                x_hbm.at[idx_v.at[pl.ds(c2 * _GCHUNK, _GCHUNK)]], buf, sem
            ).wait()
            pltpu.sync_copy(buf, o_hbm.at[pl.ds(base + c2 * _GCHUNK, _GCHUNK)])

    k = pl.kernel(
        body,
        out_type=jax.ShapeDtypeStruct((B_T, D), jnp.int32),
        mesh=_SC_MESH,
        scratch_types=[
            pltpu.VMEM((_GROWS,), jnp.int32),
            pltpu.VMEM((_GCHUNK, D), jnp.int32),
            pltpu.SemaphoreType.DMA,
        ],
    )
    return k(xpk, flat)


_KC = 512
_HC = 512

_INV_SQRT2 = 0.7071067811865476


def _gelu_exact(v):
    return 0.5 * v * (1.0 + lax.erf(v * _INV_SQRT2))


def _combine_body(y_ref, flat_ref, cnt_ref, bias_ref, res_ref, out_scr):
    k = pl.program_id(1)
    tok_row = flat_ref[0]
    oh = (lax.broadcasted_iota(jnp.int32, (B_T, _KC), 0) == tok_row).astype(
        jnp.bfloat16
    )
    yb = y_ref[...].astype(jnp.bfloat16)
    part = jnp.dot(oh, yb, preferred_element_type=jnp.float32)

    @pl.when(k == 0)
    def _():
        out_scr[...] = part

    @pl.when(k > 0)
    def _():
        out_scr[...] += part

    @pl.when(k == B_T // _KC - 1)
    def _():
        cnt = jnp.clip(cnt_ref[...], 1.0, None)
        res_ref[...] = _gelu_exact(out_scr[...] / cnt + bias_ref[...])


def _combine(y_all, flat3, counts, bias_int):
    return pl.pallas_call(
        _combine_body,
        grid=(2 * D // _HC, B_T // _KC),
        in_specs=[
            pl.BlockSpec((_KC, _HC), lambda h, k: (k, h)),
            pl.BlockSpec((1, 1, _KC), lambda h, k: (k, 0, 0)),
            pl.BlockSpec((B_T, 1), lambda h, k: (0, 0)),
            pl.BlockSpec((1, _HC), lambda h, k: (0, h)),
        ],
        out_specs=pl.BlockSpec((B_T, _HC), lambda h, k: (0, h)),
        out_shape=jax.ShapeDtypeStruct((B_T, 2 * D), jnp.float32),
        scratch_shapes=[pltpu.VMEM((B_T, _HC), jnp.float32)],
    )(y_all, flat3, counts, bias_int)


def _expert_body(xg_ref, w_ref, s_ref, y_ref):
    u = xg_ref[...]
    xr = lax.bitcast_convert_type(u << 16, jnp.float32).astype(jnp.bfloat16)
    xi = lax.bitcast_convert_type(
        u & jnp.int32(-65536), jnp.float32
    ).astype(jnp.bfloat16)
    xc = jnp.concatenate([xr, xi], axis=0)
    wi = w_ref[...].astype(jnp.int32)
    bits = ((wi << 16) & jnp.int32(-2147483648)) | (
        (wi << 13) & jnp.int32(0x0FFFE000)
    )
    wf = lax.bitcast_convert_type(bits, jnp.float32) * jnp.float32(
        5.192296858534828e33
    )
    w = wf.astype(jnp.bfloat16)
    ab = jnp.dot(xc, w, preferred_element_type=jnp.float32)
    a = ab[:K]
    b = ab[K:]
    rm1 = pltpu.roll(b, 2 * D - 1, axis=1)
    r1 = pltpu.roll(b, 1, axis=1)
    lane = lax.broadcasted_iota(jnp.int32, (K, 2 * D), 1)
    bswap = jnp.where(lane % 2 == 0, -rm1, r1)
    y_ref[...] = (a + bswap) * s_ref[...]


def _experts(xg, w3, sflat):
    return pl.pallas_call(
        _expert_body,
        grid=(E,),
        in_specs=[
            pl.BlockSpec((K, D), lambda a: (a, 0)),
            pl.BlockSpec((D, 2 * D), lambda a: (a, 0)),
            pl.BlockSpec((K, 1), lambda a: (a, 0)),
        ],
        out_specs=pl.BlockSpec((K, 2 * D), lambda a: (a, 0)),
        out_shape=jax.ShapeDtypeStruct((B_T, 2 * D), jnp.float32),
    )(xg, w3, sflat)


def kernel(x, gate_weights, experts_weight, act_bias):
    x2d = x.reshape(B_T, 2 * D)
    wu = lax.bitcast_convert_type(experts_weight, jnp.uint16).reshape(
        E * D, 2 * D
    )

    xpk = lax.bitcast_convert_type(x.astype(jnp.bfloat16), jnp.int32)

    sv, si, counts = _gate_topk(x2d, gate_weights)
    topk_scores = sv.T
    topk_indices = si.T
    flat = si.reshape(-1)

    xg = _sc_gather(xpk, flat)

    y_all = _experts(xg, wu, sv.reshape(B_T, 1))

    res2d = _combine(
        y_all,
        flat.reshape(B_T // _KC, 1, _KC),
        counts,
        jnp.repeat(act_bias, 2).reshape(1, 2 * D),
    )
    res = res2d.reshape(B_T, D, 2)
    return (res, topk_indices, topk_scores, counts.reshape(B_T, 1, 1))
